# speedup vs baseline: 1.5551x; 1.5551x over previous
import functools

import jax
import jax.numpy as jnp
from jax import lax
from jax.experimental import pallas as pl
from jax.experimental.pallas import tpu as pltpu
from jax.experimental.pallas import tpu_sc as plsc

_B, _P, _T, _D, _H = 4096, 8, 15, 256, 4
_DH = _D // _H
_UB = 32
_NG = _UB // 8
_R = 8 * _T
_NW = 32
_UPW = _B // _NW


def _sc_scatter(gnn_flat, ids_t):
    mesh = plsc.VectorSubcoreMesh(core_axis_name="c", subcore_axis_name="s")

    @functools.partial(
        pl.kernel,
        out_type=jax.ShapeDtypeStruct((_B * _T, _D), jnp.float32),
        mesh=mesh,
        scratch_types=[
            pltpu.VMEM((_UPW,), jnp.int32),
            pltpu.VMEM((_UPW,), jnp.int32),
            pltpu.VMEM((_UPW,), jnp.int32),
            pltpu.VMEM((_UPW, _D), jnp.float32),
            pltpu.SemaphoreType.DMA,
        ],
    )
    def scat(gnn_hbm, idst_hbm, out_hbm, idsv, srcv, dstv, rows, sem):
        wid = lax.axis_index("s") * 2 + lax.axis_index("c")
        ubase = wid * _UPW
        for p in range(_P):
            pltpu.sync_copy(idst_hbm.at[pl.ds(p * _B + ubase, _UPW)], idsv)
            for k in range(_UPW // 16):
                lane = lax.iota(jnp.int32, 16)
                u = ubase + k * 16 + lane
                srcv[pl.ds(k * 16, 16)] = u * _P + p
                dstv[pl.ds(k * 16, 16)] = u * _T + idsv[pl.ds(k * 16, 16)]
            pltpu.async_copy(gnn_hbm.at[srcv], rows, sem).wait()
            pltpu.async_copy(rows, out_hbm.at[dstv], sem).wait()

    return scat(gnn_flat, ids_t)


def _tc_body(buf_ref, idsc_ref, ids2_ref, pos_ref, wq_ref, wk_ref, wv_ref,
             wo_ref, bq_ref, bk_ref, bv_ref, bo_ref, gam_ref, bet_ref,
             logits_ref, last_ref, am_ref, umax_ref):
    f32 = jnp.float32
    ids2 = ids2_ref[...]
    umax = jnp.max(ids2, axis=1, keepdims=True)
    umax_ref[...] = umax
    pos = pos_ref[...]
    pos_tile = jnp.concatenate([pos] * 8, axis=0)
    idsc = idsc_ref[...].astype(f32)
    ones_r = jnp.ones((_R, 1), f32)

    ri = lax.broadcasted_iota(jnp.int32, (_R, _R), 0) // _T
    ci = lax.broadcasted_iota(jnp.int32, (_R, _R), 1) // _T
    bd = jnp.where(ri == ci, 0.0, -1e9).astype(f32)

    r64 = lax.broadcasted_iota(jnp.int32, (_R, 64), 0)
    c64 = lax.broadcasted_iota(jnp.int32, (_R, 64), 1)
    u_r = r64 // _T
    t_r = (r64 - u_r * _T).astype(f32)
    same_u = u_r == (c64 >> 3)

    xs, annegs = [], []
    for g in range(_NG):
        idsg = lax.slice(idsc, (g * 64, 0), (g * 64 + 64, 1))
        ids_b = lax.dot_general(ones_r, idsg,
                                (((1,), (1,)), ((), ())))
        hit = jnp.where(same_u & (ids_b == t_r), 1.0, 0.0)
        act = jnp.max(hit, axis=1, keepdims=True)
        annegs.append((act - 1.0) * 10000.0)
        bufg = buf_ref[pl.ds(g * _R, _R), :]
        xs.append(jnp.where(act > 0.5, bufg, 0.0) + pos_tile)
    x = jnp.concatenate(xs, axis=0)
    am_ref[...] = jnp.concatenate(annegs, axis=0)

    ctx_heads = []
    for h in range(_H):
        q = jnp.dot(x, wq_ref[pl.ds(h * _D, _D), :],
                    preferred_element_type=f32) + bq_ref[h:h + 1, :]
        k = jnp.dot(x, wk_ref[pl.ds(h * _D, _D), :],
                    preferred_element_type=f32) + bk_ref[h:h + 1, :]
        v = jnp.dot(x, wv_ref[pl.ds(h * _D, _D), :],
                    preferred_element_type=f32) + bv_ref[h:h + 1, :]
        cgs = []
        for g in range(_NG):
            qg = lax.slice(q, (g * _R, 0), (g * _R + _R, _DH))
            kg = lax.slice(k, (g * _R, 0), (g * _R + _R, _DH))
            vg = lax.slice(v, (g * _R, 0), (g * _R + _R, _DH))
            amg = lax.dot_general(ones_r, annegs[g],
                                  (((1,), (1,)), ((), ())))
            s = lax.dot_general(qg, kg, (((1,), (1,)), ((), ())),
                                preferred_element_type=f32)
            s = s * 0.125 + amg + bd
            s = s - jnp.max(s, axis=1, keepdims=True)
            e = jnp.exp(s)
            probs = e / jnp.sum(e, axis=1, keepdims=True)
            cgs.append(jnp.dot(probs, vg, preferred_element_type=f32))
        ctx_heads.append(jnp.concatenate(cgs, axis=0))

    hres = bo_ref[...] + x
    for h in range(_H):
        hres = hres + jnp.dot(ctx_heads[h], wo_ref[pl.ds(h * _DH, _DH), :],
                              preferred_element_type=f32)
    mu = jnp.mean(hres, axis=1, keepdims=True)
    dev = hres - mu
    var = jnp.mean(dev * dev, axis=1, keepdims=True)
    logits = gam_ref[...] * (dev / jnp.sqrt(var + 1e-12)) + bet_ref[...]
    logits_ref[...] = logits

    umf = umax.astype(f32)
    j8 = lax.broadcasted_iota(jnp.int32, (8, _R), 1).astype(f32)
    u8 = lax.broadcasted_iota(jnp.int32, (8, _R), 0).astype(f32)
    lasts = []
    for g in range(_NG):
        umg = lax.slice(umf, (g * 8, 0), (g * 8 + 8, 1))
        um_b = lax.dot_general(umg, ones_r, (((1,), (1,)), ((), ())))
        one_hot = jnp.where(j8 == u8 * _T + um_b, 1.0, 0.0)
        lg = lax.slice(logits, (g * _R, 0), (g * _R + _R, _D))
        lasts.append(jnp.dot(one_hot, lg, preferred_element_type=f32))
    last_ref[...] = jnp.concatenate(lasts, axis=0)


def _tc_specs():
    grid = (_B // _UB,)
    rows = _UB * _T
    in_specs = [
        pl.BlockSpec((rows, _D), lambda i: (i, 0)),
        pl.BlockSpec((_UB * _P, 1), lambda i: (i, 0)),
        pl.BlockSpec((_UB, _P), lambda i: (i, 0)),
        pl.BlockSpec((_T, _D), lambda i: (0, 0)),
        pl.BlockSpec((_H * _D, _DH), lambda i: (0, 0)),
        pl.BlockSpec((_H * _D, _DH), lambda i: (0, 0)),
        pl.BlockSpec((_H * _D, _DH), lambda i: (0, 0)),
        pl.BlockSpec((_D, _D), lambda i: (0, 0)),
        pl.BlockSpec((_H, _DH), lambda i: (0, 0)),
        pl.BlockSpec((_H, _DH), lambda i: (0, 0)),
        pl.BlockSpec((_H, _DH), lambda i: (0, 0)),
        pl.BlockSpec((1, _D), lambda i: (0, 0)),
        pl.BlockSpec((1, _D), lambda i: (0, 0)),
        pl.BlockSpec((1, _D), lambda i: (0, 0)),
    ]
    out_specs = [
        pl.BlockSpec((rows, _D), lambda i: (i, 0)),
        pl.BlockSpec((_UB, _D), lambda i: (i, 0)),
        pl.BlockSpec((rows, 1), lambda i: (i, 0)),
        pl.BlockSpec((_UB, 1), lambda i: (i, 0)),
    ]
    out_shapes = [
        jax.ShapeDtypeStruct((_B * _T, _D), jnp.float32),
        jax.ShapeDtypeStruct((_B, _D), jnp.float32),
        jax.ShapeDtypeStruct((_B * _T, 1), jnp.float32),
        jax.ShapeDtypeStruct((_B, 1), jnp.int32),
    ]
    return grid, in_specs, out_specs, out_shapes


def _tc_attention(buf, ids_col, period_ids, pos_emb, wq_r, wk_r, wv_r, Wo,
                  bq_r, bk_r, bv_r, bo_r, gam_r, bet_r):
    grid, in_specs, out_specs, out_shapes = _tc_specs()
    return pl.pallas_call(
        _tc_body,
        grid=grid,
        in_specs=in_specs,
        out_specs=out_specs,
        out_shape=out_shapes,
    )(buf, ids_col, period_ids, pos_emb, wq_r, wk_r, wv_r, Wo,
      bq_r, bk_r, bv_r, bo_r, gam_r, bet_r)


def kernel(gnn_out, period_ids, pos_emb, Wq, bq, Wk, bk, Wv, bv, Wo, bo,
           ln_gamma, ln_beta):
    gnn_flat = gnn_out.reshape(_B * _P, _D)
    ids_t = period_ids.T.reshape(-1)
    buf = _sc_scatter(gnn_flat, ids_t)

    ids_col = period_ids.reshape(_B * _P, 1)

    def per_head(w):
        return w.reshape(_D, _H, _DH).transpose(1, 0, 2).reshape(_H * _D, _DH)

    logits_f, last, am_f, umax_c = _tc_attention(
        buf, ids_col, period_ids, pos_emb,
        per_head(Wq), per_head(Wk), per_head(Wv), Wo,
        bq.reshape(_H, _DH), bk.reshape(_H, _DH), bv.reshape(_H, _DH),
        bo.reshape(1, _D), ln_gamma.reshape(1, _D), ln_beta.reshape(1, _D))

    logits = logits_f.reshape(_B, _T, _D)
    attention_mask = am_f.reshape(_B, _T)
    user_max_period = umax_c.reshape(_B)
    return logits, last, attention_mask, user_max_period


# speedup vs baseline: 1.7908x; 1.1515x over previous
import functools

import jax
import jax.numpy as jnp
from jax import lax
from jax.experimental import pallas as pl
from jax.experimental.pallas import tpu as pltpu
from jax.experimental.pallas import tpu_sc as plsc

_B, _P, _T, _D, _H = 4096, 8, 15, 256, 4
_DH = _D // _H
_UB = 64
_NG = _UB // 8
_R = 8 * _T
_NW = 32
_UPW = _B // _NW


def _sc_scatter(gnn_flat, ids_t):
    mesh = plsc.VectorSubcoreMesh(core_axis_name="c", subcore_axis_name="s")

    @functools.partial(
        pl.kernel,
        out_type=jax.ShapeDtypeStruct((_B * _T, _D), jnp.float32),
        mesh=mesh,
        scratch_types=[
            pltpu.VMEM((_P, _UPW), jnp.int32),
            pltpu.VMEM((_UPW,), jnp.int32),
            pltpu.VMEM((_P, _UPW), jnp.int32),
            pltpu.VMEM((2, _UPW, _D), jnp.float32),
            pltpu.SemaphoreType.DMA,
            pltpu.SemaphoreType.DMA,
        ],
    )
    def scat(gnn_hbm, idst_hbm, out_hbm, idsv, srcv, dstv, rows, gsem, ssem):
        wid = lax.axis_index("s") * 2 + lax.axis_index("c")
        ubase = wid * _UPW
        for p in range(_P):
            pltpu.sync_copy(idst_hbm.at[pl.ds(p * _B + ubase, _UPW)],
                            idsv.at[p])
            for k in range(_UPW // 16):
                lane = lax.iota(jnp.int32, 16)
                u = ubase + k * 16 + lane
                dstv[p, pl.ds(k * 16, 16)] = u * _T + idsv[p, pl.ds(k * 16, 16)]
        gathers = []
        for p in range(2):
            for k in range(_UPW // 16):
                lane = lax.iota(jnp.int32, 16)
                srcv[pl.ds(k * 16, 16)] = (ubase + k * 16 + lane) * _P + p
            gathers.append(pltpu.async_copy(gnn_hbm.at[srcv], rows.at[p % 2],
                                            gsem))
        for p in range(_P):
            gathers.pop(0).wait()
            cp = pltpu.async_copy(rows.at[p % 2], out_hbm.at[dstv.at[p]], ssem)
            cp.wait()
            if p + 2 < _P:
                for k in range(_UPW // 16):
                    lane = lax.iota(jnp.int32, 16)
                    srcv[pl.ds(k * 16, 16)] = (ubase + k * 16 + lane) * _P + (p + 2)
                gathers.append(pltpu.async_copy(gnn_hbm.at[srcv],
                                                rows.at[p % 2], gsem))

    return scat(gnn_flat, ids_t)


def _tc_body(buf_ref, idsc_ref, ids2_ref, pos_ref, wq_ref, wk_ref, wv_ref,
             wo_ref, bq_ref, bk_ref, bv_ref, bo_ref, gam_ref, bet_ref,
             logits_ref, last_ref, am_ref, umax_ref):
    f32 = jnp.float32
    ids2 = ids2_ref[...]
    umax = jnp.max(ids2, axis=1, keepdims=True)
    umax_ref[...] = umax
    pos_tile = pos_ref[...]
    idsc = idsc_ref[...].astype(f32)
    ones_r = jnp.ones((_R, 1), f32)

    ri = lax.broadcasted_iota(jnp.int32, (_R, _R), 0) // _T
    ci = lax.broadcasted_iota(jnp.int32, (_R, _R), 1) // _T
    bd = jnp.where(ri == ci, 0.0, -1e9).astype(f32)

    r64 = lax.broadcasted_iota(jnp.int32, (_R, 64), 0)
    c64 = lax.broadcasted_iota(jnp.int32, (_R, 64), 1)
    u_r = r64 // _T
    t_r = (r64 - u_r * _T).astype(f32)
    same_u = u_r == (c64 >> 3)

    xs, annegs, mbias = [], [], []
    for g in range(_NG):
        idsg = lax.slice(idsc, (g * 64, 0), (g * 64 + 64, 1))
        ids_b = lax.dot_general(ones_r, idsg,
                                (((1,), (1,)), ((), ())))
        hit = jnp.where(same_u & (ids_b == t_r), 1.0, 0.0)
        act = jnp.max(hit, axis=1, keepdims=True)
        anneg = (act - 1.0) * 10000.0
        annegs.append(anneg)
        mbias.append(lax.dot_general(ones_r, anneg,
                                     (((1,), (1,)), ((), ()))) + bd)
        bufg = buf_ref[pl.ds(g * _R, _R), :]
        xs.append(jnp.where(act > 0.5, bufg, 0.0) + pos_tile)
    x = jnp.concatenate(xs, axis=0)
    am_ref[...] = jnp.concatenate(annegs, axis=0)

    ctx_heads = []
    for h in range(_H):
        q = jnp.dot(x, wq_ref[pl.ds(h * _D, _D), :],
                    preferred_element_type=f32) + bq_ref[h:h + 1, :]
        k = jnp.dot(x, wk_ref[pl.ds(h * _D, _D), :],
                    preferred_element_type=f32) + bk_ref[h:h + 1, :]
        v = jnp.dot(x, wv_ref[pl.ds(h * _D, _D), :],
                    preferred_element_type=f32) + bv_ref[h:h + 1, :]
        cgs = []
        for g in range(_NG):
            qg = lax.slice(q, (g * _R, 0), (g * _R + _R, _DH))
            kg = lax.slice(k, (g * _R, 0), (g * _R + _R, _DH))
            vg = lax.slice(v, (g * _R, 0), (g * _R + _R, _DH))
            s = lax.dot_general(qg, kg, (((1,), (1,)), ((), ())),
                                preferred_element_type=f32)
            s = s + mbias[g]
            s = s - jnp.max(s, axis=1, keepdims=True)
            e = jnp.exp(s)
            probs = e / jnp.sum(e, axis=1, keepdims=True)
            cgs.append(jnp.dot(probs, vg, preferred_element_type=f32))
        ctx_heads.append(jnp.concatenate(cgs, axis=0))

    hres = bo_ref[...] + x
    for h in range(_H):
        hres = hres + jnp.dot(ctx_heads[h], wo_ref[pl.ds(h * _DH, _DH), :],
                              preferred_element_type=f32)
    mu = jnp.mean(hres, axis=1, keepdims=True)
    dev = hres - mu
    var = jnp.mean(dev * dev, axis=1, keepdims=True)
    logits = gam_ref[...] * (dev / jnp.sqrt(var + 1e-12)) + bet_ref[...]
    logits_ref[...] = logits

    umf = umax.astype(f32)
    j8 = lax.broadcasted_iota(jnp.int32, (8, _R), 1).astype(f32)
    u8 = lax.broadcasted_iota(jnp.int32, (8, _R), 0).astype(f32)
    lasts = []
    for g in range(_NG):
        umg = lax.slice(umf, (g * 8, 0), (g * 8 + 8, 1))
        um_b = lax.dot_general(umg, ones_r, (((1,), (1,)), ((), ())))
        one_hot = jnp.where(j8 == u8 * _T + um_b, 1.0, 0.0)
        lg = lax.slice(logits, (g * _R, 0), (g * _R + _R, _D))
        lasts.append(jnp.dot(one_hot, lg, preferred_element_type=f32))
    last_ref[...] = jnp.concatenate(lasts, axis=0)


def _tc_specs():
    grid = (_B // _UB,)
    rows = _UB * _T
    in_specs = [
        pl.BlockSpec((rows, _D), lambda i: (i, 0)),
        pl.BlockSpec((_UB * _P, 1), lambda i: (i, 0)),
        pl.BlockSpec((_UB, _P), lambda i: (i, 0)),
        pl.BlockSpec((_R, _D), lambda i: (0, 0)),
        pl.BlockSpec((_H * _D, _DH), lambda i: (0, 0)),
        pl.BlockSpec((_H * _D, _DH), lambda i: (0, 0)),
        pl.BlockSpec((_H * _D, _DH), lambda i: (0, 0)),
        pl.BlockSpec((_D, _D), lambda i: (0, 0)),
        pl.BlockSpec((_H, _DH), lambda i: (0, 0)),
        pl.BlockSpec((_H, _DH), lambda i: (0, 0)),
        pl.BlockSpec((_H, _DH), lambda i: (0, 0)),
        pl.BlockSpec((1, _D), lambda i: (0, 0)),
        pl.BlockSpec((1, _D), lambda i: (0, 0)),
        pl.BlockSpec((1, _D), lambda i: (0, 0)),
    ]
    out_specs = [
        pl.BlockSpec((rows, _D), lambda i: (i, 0)),
        pl.BlockSpec((_UB, _D), lambda i: (i, 0)),
        pl.BlockSpec((rows, 1), lambda i: (i, 0)),
        pl.BlockSpec((_UB, 1), lambda i: (i, 0)),
    ]
    out_shapes = [
        jax.ShapeDtypeStruct((_B * _T, _D), jnp.float32),
        jax.ShapeDtypeStruct((_B, _D), jnp.float32),
        jax.ShapeDtypeStruct((_B * _T, 1), jnp.float32),
        jax.ShapeDtypeStruct((_B, 1), jnp.int32),
    ]
    return grid, in_specs, out_specs, out_shapes


def _tc_attention(buf, ids_col, period_ids, pos_emb, wq_r, wk_r, wv_r, Wo,
                  bq_r, bk_r, bv_r, bo_r, gam_r, bet_r):
    grid, in_specs, out_specs, out_shapes = _tc_specs()
    return pl.pallas_call(
        _tc_body,
        grid=grid,
        in_specs=in_specs,
        out_specs=out_specs,
        out_shape=out_shapes,
    )(buf, ids_col, period_ids, pos_emb, wq_r, wk_r, wv_r, Wo,
      bq_r, bk_r, bv_r, bo_r, gam_r, bet_r)


def kernel(gnn_out, period_ids, pos_emb, Wq, bq, Wk, bk, Wv, bv, Wo, bo,
           ln_gamma, ln_beta):
    gnn_flat = gnn_out.reshape(_B * _P, _D)
    buf = _sc_scatter(gnn_flat, period_ids.T.reshape(-1))

    ids_col = period_ids.reshape(_B * _P, 1)
    pos_tiled = jnp.tile(pos_emb, (8, 1))

    def per_head(w):
        return w.reshape(_D, _H, _DH).transpose(1, 0, 2).reshape(_H * _D, _DH)

    logits_f, last, am_f, umax_c = _tc_attention(
        buf, ids_col, period_ids, pos_tiled,
        per_head(Wq) * 0.125, per_head(Wk), per_head(Wv), Wo,
        (bq * 0.125).reshape(_H, _DH), bk.reshape(_H, _DH),
        bv.reshape(_H, _DH),
        bo.reshape(1, _D), ln_gamma.reshape(1, _D), ln_beta.reshape(1, _D))

    logits = logits_f.reshape(_B, _T, _D)
    attention_mask = am_f.reshape(_B, _T)
    user_max_period = umax_c.reshape(_B)
    return logits, last, attention_mask, user_max_period
